# Initial kernel scaffold; baseline (speedup 1.0000x reference)
import jax
import jax.numpy as jnp
from jax.experimental import pallas as pl


def kernel(x, edge_index, W1, b1, W2, b2, Wc1, bc1, Wc2, bc2):
    raise NotImplementedError("write your pallas kernel here")


# speedup vs baseline: 27.7757x; 27.7757x over previous
import functools

import jax
import jax.numpy as jnp
from jax import lax
from jax.experimental import pallas as pl
from jax.experimental.pallas import tpu as pltpu
from jax.experimental.pallas import tpu_sc as plsc

N = 50000
E = 1600000
NC = 2
NS = 16

_mesh = plsc.VectorSubcoreMesh(core_axis_name="c", subcore_axis_name="s")
_sc_params = pltpu.CompilerParams(use_tc_tiling_on_sc=False,
                                  needs_layout_passes=False)


_DEG_C = 2000
_DEG_EPW = E // (NC * NS)


@functools.partial(
    pl.kernel,
    out_type=jax.ShapeDtypeStruct((NC * N,), jnp.float32),
    mesh=_mesh,
    compiler_params=_sc_params,
    scratch_types=[
        pltpu.VMEM((_DEG_C,), jnp.int32),
        pltpu.VMEM((_DEG_C,), jnp.float32),
        pltpu.VMEM((5000,), jnp.float32),
        pltpu.VMEM_SHARED((N,), jnp.float32),
    ],
)
def _deg_kernel(dst_hbm, out_hbm, idx_v, ones_v, zero_v, acc_sh):
    c = lax.axis_index("c")
    s = lax.axis_index("s")

    @pl.loop(0, _DEG_C, step=16)
    def _(k):
        ones_v[pl.ds(k, 16)] = jnp.ones((16,), jnp.float32)

    @pl.loop(0, 5000, step=16)
    def _(k):
        zero_v[pl.ds(k, 16)] = jnp.zeros((16,), jnp.float32)

    @pl.when(s < 10)
    def _():
        pltpu.sync_copy(zero_v, acc_sh.at[pl.ds(s * 5000, 5000)])

    plsc.subcore_barrier()

    base = (c * NS + s) * _DEG_EPW

    @pl.loop(0, _DEG_EPW, step=_DEG_C)
    def _(i):
        pltpu.sync_copy(dst_hbm.at[pl.ds(base + i, _DEG_C)], idx_v)
        pltpu.sync_copy(ones_v, acc_sh.at[idx_v], add=True)

    plsc.subcore_barrier()

    @pl.when(s < 10)
    def _():
        pltpu.sync_copy(acc_sh.at[pl.ds(s * 5000, 5000)], zero_v)
        pltpu.sync_copy(zero_v, out_hbm.at[pl.ds(c * N + s * 5000, 5000)])


def _make_conv_scatter(NPC, C):
    H = 16
    EPW = E // NS
    ZR = 1000
    NZCH = N // ZR
    NQ = NC * NPC

    @functools.partial(
        pl.kernel,
        out_type=tuple(jax.ShapeDtypeStruct((N, H), jnp.float32)
                       for _ in range(NQ)),
        mesh=_mesh,
        compiler_params=_sc_params,
        scratch_types=[
            pltpu.VMEM((C,), jnp.int32),
            pltpu.VMEM((C,), jnp.int32),
            pltpu.VMEM((C, H), jnp.float32),
            pltpu.VMEM((ZR, H), jnp.float32),
            pltpu.VMEM_SHARED((N, H), jnp.float32),
        ],
    )
    def conv_kernel(src_hbm, dst_hbm, *refs):
        y_refs = refs[:NQ]
        out_refs = refs[NQ:2 * NQ]
        si_v, di_v, rows_v, zero_v, acc_sh = refs[2 * NQ:]
        c = lax.axis_index("c")
        s = lax.axis_index("s")

        @pl.loop(0, ZR)
        def _(r):
            zero_v[r, pl.ds(0, 16)] = jnp.zeros((16,), jnp.float32)

        base = s * EPW

        def run_pass(y_hbm, out_hbm):
            for j in range((NZCH + NS - 1) // NS):
                k = s + j * NS

                @pl.when(k < NZCH)
                def _():
                    pltpu.sync_copy(zero_v, acc_sh.at[pl.ds(k * ZR, ZR)])

            plsc.subcore_barrier()

            @pl.loop(0, EPW, step=C)
            def _(i):
                pltpu.sync_copy(src_hbm.at[pl.ds(base + i, C)], si_v)
                pltpu.sync_copy(dst_hbm.at[pl.ds(base + i, C)], di_v)
                pltpu.sync_copy(y_hbm.at[si_v], rows_v)
                pltpu.sync_copy(rows_v, acc_sh.at[di_v], add=True)

            plsc.subcore_barrier()

            for j in range((NZCH + NS - 1) // NS):
                k = s + j * NS

                @pl.when(k < NZCH)
                def _():
                    pltpu.sync_copy(acc_sh.at[pl.ds(k * ZR, ZR)], zero_v)
                    pltpu.sync_copy(zero_v, out_hbm.at[pl.ds(k * ZR, ZR)])

            plsc.subcore_barrier()

            @pl.loop(0, ZR)
            def _(r):
                zero_v[r, pl.ds(0, 16)] = jnp.zeros((16,), jnp.float32)

        for cv in range(NC):
            @pl.when(c == cv)
            def _(cv=cv):
                for p in range(NPC):
                    qi = cv * NPC + p
                    run_pass(y_refs[qi], out_refs[qi])

    return conv_kernel


_conv_scatter_q2 = _make_conv_scatter(2, 2000)
_conv_scatter_q1 = _make_conv_scatter(1, 2000)


_CLS_C = 2000
_CLS_EPW = E // (NC * NS)


@functools.partial(
    pl.kernel,
    out_type=jax.ShapeDtypeStruct((E,), jnp.float32),
    mesh=_mesh,
    compiler_params=_sc_params,
    scratch_types=[
        pltpu.VMEM((_CLS_C,), jnp.int32),
        pltpu.VMEM((_CLS_C,), jnp.int32),
        pltpu.VMEM((_CLS_C, 16), jnp.float32),
        pltpu.VMEM((_CLS_C, 16), jnp.float32),
        pltpu.VMEM((_CLS_C,), jnp.float32),
        pltpu.VMEM((16,), jnp.float32),
        pltpu.VMEM((16,), jnp.float32),
    ],
)
def _cls_kernel(src_hbm, dst_hbm, p_hbm, q_hbm, w_hbm, b_hbm, out_hbm,
                si_v, di_v, pa_v, qa_v, o_v, w_v, b_v):
    c = lax.axis_index("c")
    s = lax.axis_index("s")

    pltpu.sync_copy(w_hbm, w_v)
    pltpu.sync_copy(b_hbm, b_v)
    wvec = w_v[...]
    ws = [wvec[f] for f in range(16)]
    bc2v = b_v[...]

    base = (c * NS + s) * _CLS_EPW

    @pl.loop(0, _CLS_EPW, step=_CLS_C)
    def _(i):
        pltpu.sync_copy(src_hbm.at[pl.ds(base + i, _CLS_C)], si_v)
        pltpu.sync_copy(dst_hbm.at[pl.ds(base + i, _CLS_C)], di_v)
        pltpu.sync_copy(p_hbm.at[si_v], pa_v)
        pltpu.sync_copy(q_hbm.at[di_v], qa_v)

        @pl.loop(0, _CLS_C // 16)
        def _(t):
            rowi = t * 16 + lax.iota(jnp.int32, 16)
            acc = jnp.zeros((16,), jnp.float32)
            for f in range(16):
                colf = jnp.full((16,), f, jnp.int32)
                av = plsc.load_gather(pa_v, [rowi, colf])
                bv = plsc.load_gather(qa_v, [rowi, colf])
                acc = acc + jnp.maximum(av + bv, 0.0) * ws[f]
            logit = acc + bc2v
            o_v[pl.ds(t * 16, 16)] = 1.0 / (1.0 + jnp.exp(-logit))

        pltpu.sync_copy(o_v, out_hbm.at[pl.ds(base + i, _CLS_C)])


_R = 2000
_G = N // _R


def _tc_xw1(x, W1):
    def body(x_ref, w_ref, o_ref):
        o_ref[...] = jnp.dot(x_ref[...], w_ref[...],
                             preferred_element_type=jnp.float32)

    return pl.pallas_call(
        body,
        grid=(_G,),
        in_specs=[pl.BlockSpec((_R, 9), lambda i: (i, 0)),
                  pl.BlockSpec((9, 64), lambda i: (0, 0))],
        out_specs=pl.BlockSpec((_R, 64), lambda i: (i, 0)),
        out_shape=jax.ShapeDtypeStruct((N, 64), jnp.float32),
    )(x, W1)


def _tc_prep1(deg2, xw1):
    def body(d_ref, xw_ref, dinv_ref, y0_ref, y1_ref, y2_ref, y3_ref):
        deg = d_ref[0] + d_ref[1] + 1.0
        dv = lax.rsqrt(deg)
        y = dv * xw_ref[...]
        dinv_ref[...] = dv
        y0_ref[...] = y[:, 0:16]
        y1_ref[...] = y[:, 16:32]
        y2_ref[...] = y[:, 32:48]
        y3_ref[...] = y[:, 48:64]

    return pl.pallas_call(
        body,
        grid=(_G,),
        in_specs=[pl.BlockSpec((2, _R, 1), lambda i: (0, i, 0)),
                  pl.BlockSpec((_R, 64), lambda i: (i, 0))],
        out_specs=[pl.BlockSpec((_R, 1), lambda i: (i, 0))]
        + [pl.BlockSpec((_R, 16), lambda i: (i, 0))] * 4,
        out_shape=[jax.ShapeDtypeStruct((N, 1), jnp.float32)]
        + [jax.ShapeDtypeStruct((N, 16), jnp.float32)] * 4,
    )(deg2, xw1)


def _tc_mid(accs, ys, dinv, b1, W2):
    def body(a0, a1, a2, a3, y0, y1, y2r, y3, dv_ref, b_ref, w_ref,
             oa_ref, ob_ref):
        dv = dv_ref[...]
        b = b_ref[...]
        hs = [jnp.maximum(dv * (a[...] + y[...]) + b[:, 16 * q:16 * (q + 1)],
                          0.0)
              for q, (a, y) in enumerate(zip((a0, a1, a2, a3),
                                             (y0, y1, y2r, y3)))]
        h1 = jnp.concatenate(hs, axis=1)
        y2 = dv * jnp.dot(h1, w_ref[...], preferred_element_type=jnp.float32)
        oa_ref[...] = y2[:, :16]
        ob_ref[...] = y2[:, 16:]

    return pl.pallas_call(
        body,
        grid=(_G,),
        in_specs=[pl.BlockSpec((_R, 16), lambda i: (i, 0))] * 8
        + [pl.BlockSpec((_R, 1), lambda i: (i, 0)),
           pl.BlockSpec((1, 64), lambda i: (0, 0)),
           pl.BlockSpec((64, 32), lambda i: (0, 0))],
        out_specs=[pl.BlockSpec((_R, 16), lambda i: (i, 0)),
                   pl.BlockSpec((_R, 16), lambda i: (i, 0))],
        out_shape=[jax.ShapeDtypeStruct((N, 16), jnp.float32),
                   jax.ShapeDtypeStruct((N, 16), jnp.float32)],
    )(*accs, *ys, dinv, b1, W2)


def _tc_cls_prep(acc2a, acc2b, y2a, y2b, dinv, b2, Wc1, bc1):
    def body(aa_ref, ab_ref, ya_ref, yb_ref, dv_ref, b_ref, w_ref, bc_ref,
             p_ref, q_ref):
        dv = dv_ref[...]
        h2a = dv * (aa_ref[...] + ya_ref[...]) + b_ref[:, :16]
        h2b = dv * (ab_ref[...] + yb_ref[...]) + b_ref[:, 16:]
        h2 = jnp.concatenate([h2a, h2b], axis=1)
        w = w_ref[...]
        p_ref[...] = jnp.dot(h2, w[:32], preferred_element_type=jnp.float32) \
            + bc_ref[...]
        q_ref[...] = jnp.dot(h2, w[32:], preferred_element_type=jnp.float32)

    return pl.pallas_call(
        body,
        grid=(_G,),
        in_specs=[pl.BlockSpec((_R, 16), lambda i: (i, 0)),
                  pl.BlockSpec((_R, 16), lambda i: (i, 0)),
                  pl.BlockSpec((_R, 16), lambda i: (i, 0)),
                  pl.BlockSpec((_R, 16), lambda i: (i, 0)),
                  pl.BlockSpec((_R, 1), lambda i: (i, 0)),
                  pl.BlockSpec((1, 32), lambda i: (0, 0)),
                  pl.BlockSpec((64, 16), lambda i: (0, 0)),
                  pl.BlockSpec((1, 16), lambda i: (0, 0))],
        out_specs=[pl.BlockSpec((_R, 16), lambda i: (i, 0)),
                   pl.BlockSpec((_R, 16), lambda i: (i, 0))],
        out_shape=[jax.ShapeDtypeStruct((N, 16), jnp.float32),
                   jax.ShapeDtypeStruct((N, 16), jnp.float32)],
    )(acc2a, acc2b, y2a, y2b, dinv, b2, Wc1, bc1)


def kernel(x, edge_index, W1, b1, W2, b2, Wc1, bc1, Wc2, bc2):
    src_idx = edge_index[0]
    dst_idx = edge_index[1]
    deg2 = _deg_kernel(dst_idx)
    xw1 = _tc_xw1(x, W1)
    dinv, y10, y11, y12, y13 = _tc_prep1(deg2.reshape(2, N, 1), xw1)
    acc1s = _conv_scatter_q2(src_idx, dst_idx, y10, y11, y12, y13)
    y2a, y2b = _tc_mid(acc1s, (y10, y11, y12, y13), dinv,
                       b1.reshape(1, 64), W2)
    acc2a, acc2b = _conv_scatter_q1(src_idx, dst_idx, y2a, y2b)
    p, q = _tc_cls_prep(acc2a, acc2b, y2a, y2b, dinv,
                        b2.reshape(1, 32), Wc1, bc1.reshape(1, 16))
    scores = _cls_kernel(src_idx, dst_idx, p, q, Wc2.reshape(16),
                         jnp.broadcast_to(bc2, (16,)))
    return scores.reshape(E, 1)


# speedup vs baseline: 31.7080x; 1.1416x over previous
import functools

import jax
import jax.numpy as jnp
from jax import lax
from jax.experimental import pallas as pl
from jax.experimental.pallas import tpu as pltpu
from jax.experimental.pallas import tpu_sc as plsc

N = 50000
E = 1600000
NC = 2
NS = 16

_mesh = plsc.VectorSubcoreMesh(core_axis_name="c", subcore_axis_name="s")
_sc_params = pltpu.CompilerParams(use_tc_tiling_on_sc=False,
                                  needs_layout_passes=False)


_DEG_C = 2000
_DEG_EPW = E // (NC * NS)


@functools.partial(
    pl.kernel,
    out_type=jax.ShapeDtypeStruct((NC * N,), jnp.float32),
    mesh=_mesh,
    compiler_params=_sc_params,
    scratch_types=[
        pltpu.VMEM((_DEG_C,), jnp.int32),
        pltpu.VMEM((_DEG_C,), jnp.float32),
        pltpu.VMEM((5000,), jnp.float32),
        pltpu.VMEM_SHARED((N,), jnp.float32),
    ],
)
def _deg_kernel(dst_hbm, out_hbm, idx_v, ones_v, zero_v, acc_sh):
    c = lax.axis_index("c")
    s = lax.axis_index("s")

    @pl.loop(0, _DEG_C, step=16)
    def _(k):
        ones_v[pl.ds(k, 16)] = jnp.ones((16,), jnp.float32)

    @pl.loop(0, 5000, step=16)
    def _(k):
        zero_v[pl.ds(k, 16)] = jnp.zeros((16,), jnp.float32)

    @pl.when(s < 10)
    def _():
        pltpu.sync_copy(zero_v, acc_sh.at[pl.ds(s * 5000, 5000)])

    plsc.subcore_barrier()

    base = (c * NS + s) * _DEG_EPW

    @pl.loop(0, _DEG_EPW, step=_DEG_C)
    def _(i):
        pltpu.sync_copy(dst_hbm.at[pl.ds(base + i, _DEG_C)], idx_v)
        pltpu.sync_copy(ones_v, acc_sh.at[idx_v], add=True)

    plsc.subcore_barrier()

    @pl.when(s < 10)
    def _():
        pltpu.sync_copy(acc_sh.at[pl.ds(s * 5000, 5000)], zero_v)
        pltpu.sync_copy(zero_v, out_hbm.at[pl.ds(c * N + s * 5000, 5000)])


def _make_conv_scatter(NPC, C):
    H = 16
    EPW = E // NS
    ZR = 1000
    NZCH = N // ZR
    NQ = NC * NPC

    @functools.partial(
        pl.kernel,
        out_type=tuple(jax.ShapeDtypeStruct((N, H), jnp.float32)
                       for _ in range(NQ)),
        mesh=_mesh,
        compiler_params=_sc_params,
        scratch_types=[
            pltpu.VMEM((C,), jnp.int32),
            pltpu.VMEM((C,), jnp.int32),
            pltpu.VMEM((C,), jnp.int32),
            pltpu.VMEM((C,), jnp.int32),
            pltpu.VMEM((C, H), jnp.float32),
            pltpu.VMEM((C, H), jnp.float32),
            pltpu.VMEM((ZR, H), jnp.float32),
            pltpu.VMEM_SHARED((N, H), jnp.float32),
        ] + [pltpu.SemaphoreType.DMA] * 8,
    )
    def conv_kernel(src_hbm, dst_hbm, *refs):
        y_refs = refs[:NQ]
        out_refs = refs[NQ:2 * NQ]
        (si0, si1, di0, di1, rows0, rows1, zero_v, acc_sh,
         gsem0, gsem1, ssem0, ssem1, isem0, isem1, dsem0, dsem1) = refs[2 * NQ:]
        c = lax.axis_index("c")
        s = lax.axis_index("s")

        @pl.loop(0, ZR)
        def _(r):
            zero_v[r, pl.ds(0, 16)] = jnp.zeros((16,), jnp.float32)

        base = s * EPW

        def run_pass(y_hbm, out_hbm):
            for j in range((NZCH + NS - 1) // NS):
                k = s + j * NS

                @pl.when(k < NZCH)
                def _():
                    pltpu.sync_copy(zero_v, acc_sh.at[pl.ds(k * ZR, ZR)])

            plsc.subcore_barrier()

            NIT = EPW // C

            def src_sl(j):
                return src_hbm.at[pl.ds(base + j * C, C)]

            def dst_sl(j):
                return dst_hbm.at[pl.ds(base + j * C, C)]

            pltpu.async_copy(dst_sl(0), di0, dsem0)
            pltpu.sync_copy(src_sl(0), si0)
            pltpu.async_copy(y_hbm.at[si0], rows0, gsem0)
            pltpu.async_copy(src_sl(1), si1, isem1)
            pltpu.async_copy(dst_sl(1), di1, dsem1)

            @pl.loop(0, NIT, step=2)
            def _(i):
                pltpu.make_async_copy(y_hbm.at[si0], rows0, gsem0).wait()
                pltpu.make_async_copy(dst_sl(i), di0, dsem0).wait()
                pltpu.async_copy(rows0, acc_sh.at[di0], ssem0, add=True)

                @pl.when(i + 2 < NIT)
                def _():
                    pltpu.async_copy(src_sl(i + 2), si0, isem0)

                @pl.when(i > 0)
                def _():
                    pltpu.make_async_copy(rows1, acc_sh.at[di1], ssem1).wait()
                    pltpu.async_copy(dst_sl(i + 1), di1, dsem1)

                pltpu.make_async_copy(src_sl(i + 1), si1, isem1).wait()
                pltpu.async_copy(y_hbm.at[si1], rows1, gsem1)
                pltpu.make_async_copy(y_hbm.at[si1], rows1, gsem1).wait()
                pltpu.make_async_copy(dst_sl(i + 1), di1, dsem1).wait()
                pltpu.async_copy(rows1, acc_sh.at[di1], ssem1, add=True)

                @pl.when(i + 3 < NIT)
                def _():
                    pltpu.async_copy(src_sl(i + 3), si1, isem1)

                pltpu.make_async_copy(rows0, acc_sh.at[di0], ssem0).wait()

                @pl.when(i + 2 < NIT)
                def _():
                    pltpu.async_copy(dst_sl(i + 2), di0, dsem0)
                    pltpu.make_async_copy(src_sl(i + 2), si0, isem0).wait()
                    pltpu.async_copy(y_hbm.at[si0], rows0, gsem0)

            pltpu.make_async_copy(rows1, acc_sh.at[di1], ssem1).wait()

            plsc.subcore_barrier()

            for j in range((NZCH + NS - 1) // NS):
                k = s + j * NS

                @pl.when(k < NZCH)
                def _():
                    pltpu.sync_copy(acc_sh.at[pl.ds(k * ZR, ZR)], zero_v)
                    pltpu.sync_copy(zero_v, out_hbm.at[pl.ds(k * ZR, ZR)])

            plsc.subcore_barrier()

            @pl.loop(0, ZR)
            def _(r):
                zero_v[r, pl.ds(0, 16)] = jnp.zeros((16,), jnp.float32)

        for cv in range(NC):
            @pl.when(c == cv)
            def _(cv=cv):
                for p in range(NPC):
                    qi = cv * NPC + p
                    run_pass(y_refs[qi], out_refs[qi])

    return conv_kernel


_conv_scatter_q2 = _make_conv_scatter(2, 1000)
_conv_scatter_q1 = _make_conv_scatter(1, 1000)


_CLS_C = 2000
_CLS_EPW = E // (NC * NS)


@functools.partial(
    pl.kernel,
    out_type=jax.ShapeDtypeStruct((E,), jnp.float32),
    mesh=_mesh,
    compiler_params=_sc_params,
    scratch_types=[
        pltpu.VMEM((_CLS_C,), jnp.int32),
        pltpu.VMEM((_CLS_C,), jnp.int32),
        pltpu.VMEM((_CLS_C, 16), jnp.float32),
        pltpu.VMEM((_CLS_C, 16), jnp.float32),
        pltpu.VMEM((_CLS_C,), jnp.float32),
        pltpu.VMEM((16,), jnp.float32),
        pltpu.VMEM((16,), jnp.float32),
    ],
)
def _cls_kernel(src_hbm, dst_hbm, p_hbm, q_hbm, w_hbm, b_hbm, out_hbm,
                si_v, di_v, pa_v, qa_v, o_v, w_v, b_v):
    c = lax.axis_index("c")
    s = lax.axis_index("s")

    pltpu.sync_copy(w_hbm, w_v)
    pltpu.sync_copy(b_hbm, b_v)
    wvec = w_v[...]
    ws = [wvec[f] for f in range(16)]
    bc2v = b_v[...]

    base = (c * NS + s) * _CLS_EPW

    @pl.loop(0, _CLS_EPW, step=_CLS_C)
    def _(i):
        pltpu.sync_copy(src_hbm.at[pl.ds(base + i, _CLS_C)], si_v)
        pltpu.sync_copy(dst_hbm.at[pl.ds(base + i, _CLS_C)], di_v)
        pltpu.sync_copy(p_hbm.at[si_v], pa_v)
        pltpu.sync_copy(q_hbm.at[di_v], qa_v)

        @pl.loop(0, _CLS_C // 16)
        def _(t):
            rowi = t * 16 + lax.iota(jnp.int32, 16)
            acc = jnp.zeros((16,), jnp.float32)
            for f in range(16):
                colf = jnp.full((16,), f, jnp.int32)
                av = plsc.load_gather(pa_v, [rowi, colf])
                bv = plsc.load_gather(qa_v, [rowi, colf])
                acc = acc + jnp.maximum(av + bv, 0.0) * ws[f]
            logit = acc + bc2v
            o_v[pl.ds(t * 16, 16)] = 1.0 / (1.0 + jnp.exp(-logit))

        pltpu.sync_copy(o_v, out_hbm.at[pl.ds(base + i, _CLS_C)])


_R = 2000
_G = N // _R


def _tc_xw1(x, W1):
    def body(x_ref, w_ref, o_ref):
        o_ref[...] = jnp.dot(x_ref[...], w_ref[...],
                             preferred_element_type=jnp.float32)

    return pl.pallas_call(
        body,
        grid=(_G,),
        in_specs=[pl.BlockSpec((_R, 9), lambda i: (i, 0)),
                  pl.BlockSpec((9, 64), lambda i: (0, 0))],
        out_specs=pl.BlockSpec((_R, 64), lambda i: (i, 0)),
        out_shape=jax.ShapeDtypeStruct((N, 64), jnp.float32),
    )(x, W1)


def _tc_prep1(deg2, xw1):
    def body(d_ref, xw_ref, dinv_ref, y0_ref, y1_ref, y2_ref, y3_ref):
        deg = d_ref[0] + d_ref[1] + 1.0
        dv = lax.rsqrt(deg)
        y = dv * xw_ref[...]
        dinv_ref[...] = dv
        y0_ref[...] = y[:, 0:16]
        y1_ref[...] = y[:, 16:32]
        y2_ref[...] = y[:, 32:48]
        y3_ref[...] = y[:, 48:64]

    return pl.pallas_call(
        body,
        grid=(_G,),
        in_specs=[pl.BlockSpec((2, _R, 1), lambda i: (0, i, 0)),
                  pl.BlockSpec((_R, 64), lambda i: (i, 0))],
        out_specs=[pl.BlockSpec((_R, 1), lambda i: (i, 0))]
        + [pl.BlockSpec((_R, 16), lambda i: (i, 0))] * 4,
        out_shape=[jax.ShapeDtypeStruct((N, 1), jnp.float32)]
        + [jax.ShapeDtypeStruct((N, 16), jnp.float32)] * 4,
    )(deg2, xw1)


def _tc_mid(accs, ys, dinv, b1, W2):
    def body(a0, a1, a2, a3, y0, y1, y2r, y3, dv_ref, b_ref, w_ref,
             oa_ref, ob_ref):
        dv = dv_ref[...]
        b = b_ref[...]
        hs = [jnp.maximum(dv * (a[...] + y[...]) + b[:, 16 * q:16 * (q + 1)],
                          0.0)
              for q, (a, y) in enumerate(zip((a0, a1, a2, a3),
                                             (y0, y1, y2r, y3)))]
        h1 = jnp.concatenate(hs, axis=1)
        y2 = dv * jnp.dot(h1, w_ref[...], preferred_element_type=jnp.float32)
        oa_ref[...] = y2[:, :16]
        ob_ref[...] = y2[:, 16:]

    return pl.pallas_call(
        body,
        grid=(_G,),
        in_specs=[pl.BlockSpec((_R, 16), lambda i: (i, 0))] * 8
        + [pl.BlockSpec((_R, 1), lambda i: (i, 0)),
           pl.BlockSpec((1, 64), lambda i: (0, 0)),
           pl.BlockSpec((64, 32), lambda i: (0, 0))],
        out_specs=[pl.BlockSpec((_R, 16), lambda i: (i, 0)),
                   pl.BlockSpec((_R, 16), lambda i: (i, 0))],
        out_shape=[jax.ShapeDtypeStruct((N, 16), jnp.float32),
                   jax.ShapeDtypeStruct((N, 16), jnp.float32)],
    )(*accs, *ys, dinv, b1, W2)


def _tc_cls_prep(acc2a, acc2b, y2a, y2b, dinv, b2, Wc1, bc1):
    def body(aa_ref, ab_ref, ya_ref, yb_ref, dv_ref, b_ref, w_ref, bc_ref,
             p_ref, q_ref):
        dv = dv_ref[...]
        h2a = dv * (aa_ref[...] + ya_ref[...]) + b_ref[:, :16]
        h2b = dv * (ab_ref[...] + yb_ref[...]) + b_ref[:, 16:]
        h2 = jnp.concatenate([h2a, h2b], axis=1)
        w = w_ref[...]
        p_ref[...] = jnp.dot(h2, w[:32], preferred_element_type=jnp.float32) \
            + bc_ref[...]
        q_ref[...] = jnp.dot(h2, w[32:], preferred_element_type=jnp.float32)

    return pl.pallas_call(
        body,
        grid=(_G,),
        in_specs=[pl.BlockSpec((_R, 16), lambda i: (i, 0)),
                  pl.BlockSpec((_R, 16), lambda i: (i, 0)),
                  pl.BlockSpec((_R, 16), lambda i: (i, 0)),
                  pl.BlockSpec((_R, 16), lambda i: (i, 0)),
                  pl.BlockSpec((_R, 1), lambda i: (i, 0)),
                  pl.BlockSpec((1, 32), lambda i: (0, 0)),
                  pl.BlockSpec((64, 16), lambda i: (0, 0)),
                  pl.BlockSpec((1, 16), lambda i: (0, 0))],
        out_specs=[pl.BlockSpec((_R, 16), lambda i: (i, 0)),
                   pl.BlockSpec((_R, 16), lambda i: (i, 0))],
        out_shape=[jax.ShapeDtypeStruct((N, 16), jnp.float32),
                   jax.ShapeDtypeStruct((N, 16), jnp.float32)],
    )(acc2a, acc2b, y2a, y2b, dinv, b2, Wc1, bc1)


def kernel(x, edge_index, W1, b1, W2, b2, Wc1, bc1, Wc2, bc2):
    src_idx = edge_index[0]
    dst_idx = edge_index[1]
    deg2 = _deg_kernel(dst_idx)
    xw1 = _tc_xw1(x, W1)
    dinv, y10, y11, y12, y13 = _tc_prep1(deg2.reshape(2, N, 1), xw1)
    acc1s = _conv_scatter_q2(src_idx, dst_idx, y10, y11, y12, y13)
    y2a, y2b = _tc_mid(acc1s, (y10, y11, y12, y13), dinv,
                       b1.reshape(1, 64), W2)
    acc2a, acc2b = _conv_scatter_q1(src_idx, dst_idx, y2a, y2b)
    p, q = _tc_cls_prep(acc2a, acc2b, y2a, y2b, dinv,
                        b2.reshape(1, 32), Wc1, bc1.reshape(1, 16))
    scores = _cls_kernel(src_idx, dst_idx, p, q, Wc2.reshape(16),
                         jnp.broadcast_to(bc2, (16,)))
    return scores.reshape(E, 1)


# speedup vs baseline: 33.9956x; 1.0721x over previous
import functools

import jax
import jax.numpy as jnp
from jax import lax
from jax.experimental import pallas as pl
from jax.experimental.pallas import tpu as pltpu
from jax.experimental.pallas import tpu_sc as plsc

N = 50000
E = 1600000
NC = 2
NS = 16

_mesh = plsc.VectorSubcoreMesh(core_axis_name="c", subcore_axis_name="s")
_sc_params = pltpu.CompilerParams(use_tc_tiling_on_sc=False,
                                  needs_layout_passes=False)


_DEG_C = 2000
_DEG_EPW = E // (NC * NS)


@functools.partial(
    pl.kernel,
    out_type=jax.ShapeDtypeStruct((NC * N,), jnp.float32),
    mesh=_mesh,
    compiler_params=_sc_params,
    scratch_types=[
        pltpu.VMEM((_DEG_C,), jnp.int32),
        pltpu.VMEM((_DEG_C,), jnp.float32),
        pltpu.VMEM((5000,), jnp.float32),
        pltpu.VMEM_SHARED((N,), jnp.float32),
    ],
)
def _deg_kernel(dst_hbm, out_hbm, idx_v, ones_v, zero_v, acc_sh):
    c = lax.axis_index("c")
    s = lax.axis_index("s")

    @pl.loop(0, _DEG_C, step=16)
    def _(k):
        ones_v[pl.ds(k, 16)] = jnp.ones((16,), jnp.float32)

    @pl.loop(0, 5000, step=16)
    def _(k):
        zero_v[pl.ds(k, 16)] = jnp.zeros((16,), jnp.float32)

    @pl.when(s < 10)
    def _():
        pltpu.sync_copy(zero_v, acc_sh.at[pl.ds(s * 5000, 5000)])

    plsc.subcore_barrier()

    base = (c * NS + s) * _DEG_EPW

    @pl.loop(0, _DEG_EPW, step=_DEG_C)
    def _(i):
        pltpu.sync_copy(dst_hbm.at[pl.ds(base + i, _DEG_C)], idx_v)
        pltpu.sync_copy(ones_v, acc_sh.at[idx_v], add=True)

    plsc.subcore_barrier()

    @pl.when(s < 10)
    def _():
        pltpu.sync_copy(acc_sh.at[pl.ds(s * 5000, 5000)], zero_v)
        pltpu.sync_copy(zero_v, out_hbm.at[pl.ds(c * N + s * 5000, 5000)])


def _make_conv_scatter(NPC, C):
    H = 16
    EPW = E // NS
    ZR = 1000
    NZCH = N // ZR
    NQ = NC * NPC

    @functools.partial(
        pl.kernel,
        out_type=tuple(jax.ShapeDtypeStruct((N, H), jnp.float32)
                       for _ in range(NQ)),
        mesh=_mesh,
        compiler_params=_sc_params,
        scratch_types=[
            pltpu.VMEM((C,), jnp.int32),
            pltpu.VMEM((C,), jnp.int32),
            pltpu.VMEM((C,), jnp.int32),
            pltpu.VMEM((C,), jnp.int32),
            pltpu.VMEM((C, H), jnp.float32),
            pltpu.VMEM((C, H), jnp.float32),
            pltpu.VMEM((ZR, H), jnp.float32),
            pltpu.VMEM_SHARED((N, H), jnp.float32),
        ] + [pltpu.SemaphoreType.DMA] * 8,
    )
    def conv_kernel(src_hbm, dst_hbm, *refs):
        y_refs = refs[:NQ]
        out_refs = refs[NQ:2 * NQ]
        (si0, si1, di0, di1, rows0, rows1, zero_v, acc_sh,
         gsem0, gsem1, ssem0, ssem1, isem0, isem1, dsem0, dsem1) = refs[2 * NQ:]
        c = lax.axis_index("c")
        s = lax.axis_index("s")

        @pl.loop(0, ZR)
        def _(r):
            zero_v[r, pl.ds(0, 16)] = jnp.zeros((16,), jnp.float32)

        base = s * EPW

        def run_pass(y_hbm, out_hbm):
            for j in range((NZCH + NS - 1) // NS):
                k = s + j * NS

                @pl.when(k < NZCH)
                def _():
                    pltpu.sync_copy(zero_v, acc_sh.at[pl.ds(k * ZR, ZR)])

            plsc.subcore_barrier()

            NIT = EPW // C

            def src_sl(j):
                return src_hbm.at[pl.ds(base + j * C, C)]

            def dst_sl(j):
                return dst_hbm.at[pl.ds(base + j * C, C)]

            pltpu.async_copy(dst_sl(0), di0, dsem0)
            pltpu.sync_copy(src_sl(0), si0)
            pltpu.async_copy(y_hbm.at[si0], rows0, gsem0)
            pltpu.async_copy(src_sl(1), si1, isem1)
            pltpu.async_copy(dst_sl(1), di1, dsem1)

            @pl.loop(0, NIT, step=2)
            def _(i):
                pltpu.make_async_copy(y_hbm.at[si0], rows0, gsem0).wait()
                pltpu.make_async_copy(dst_sl(i), di0, dsem0).wait()
                pltpu.async_copy(rows0, acc_sh.at[di0], ssem0, add=True)

                @pl.when(i + 2 < NIT)
                def _():
                    pltpu.async_copy(src_sl(i + 2), si0, isem0)

                @pl.when(i > 0)
                def _():
                    pltpu.make_async_copy(rows1, acc_sh.at[di1], ssem1).wait()
                    pltpu.async_copy(dst_sl(i + 1), di1, dsem1)

                pltpu.make_async_copy(src_sl(i + 1), si1, isem1).wait()
                pltpu.async_copy(y_hbm.at[si1], rows1, gsem1)
                pltpu.make_async_copy(y_hbm.at[si1], rows1, gsem1).wait()
                pltpu.make_async_copy(dst_sl(i + 1), di1, dsem1).wait()
                pltpu.async_copy(rows1, acc_sh.at[di1], ssem1, add=True)

                @pl.when(i + 3 < NIT)
                def _():
                    pltpu.async_copy(src_sl(i + 3), si1, isem1)

                pltpu.make_async_copy(rows0, acc_sh.at[di0], ssem0).wait()

                @pl.when(i + 2 < NIT)
                def _():
                    pltpu.async_copy(dst_sl(i + 2), di0, dsem0)
                    pltpu.make_async_copy(src_sl(i + 2), si0, isem0).wait()
                    pltpu.async_copy(y_hbm.at[si0], rows0, gsem0)

            pltpu.make_async_copy(rows1, acc_sh.at[di1], ssem1).wait()

            plsc.subcore_barrier()

            for j in range((NZCH + NS - 1) // NS):
                k = s + j * NS

                @pl.when(k < NZCH)
                def _():
                    pltpu.sync_copy(acc_sh.at[pl.ds(k * ZR, ZR)], zero_v)
                    pltpu.sync_copy(zero_v, out_hbm.at[pl.ds(k * ZR, ZR)])

            plsc.subcore_barrier()

            @pl.loop(0, ZR)
            def _(r):
                zero_v[r, pl.ds(0, 16)] = jnp.zeros((16,), jnp.float32)

        for cv in range(NC):
            @pl.when(c == cv)
            def _(cv=cv):
                for p in range(NPC):
                    qi = cv * NPC + p
                    run_pass(y_refs[qi], out_refs[qi])

    return conv_kernel


_conv_scatter_q2 = _make_conv_scatter(2, 1000)
_conv_scatter_q1 = _make_conv_scatter(1, 1000)


_CLS_C = 400
_CLS_EPW = E // (NC * NS)


@functools.partial(
    pl.kernel,
    out_type=jax.ShapeDtypeStruct((E,), jnp.float32),
    mesh=_mesh,
    compiler_params=_sc_params,
    scratch_types=[
        pltpu.VMEM((_CLS_C,), jnp.int32),
        pltpu.VMEM((_CLS_C,), jnp.int32),
        pltpu.VMEM((_CLS_C,), jnp.int32),
        pltpu.VMEM((_CLS_C,), jnp.int32),
        pltpu.VMEM((_CLS_C, 16), jnp.float32),
        pltpu.VMEM((_CLS_C, 16), jnp.float32),
        pltpu.VMEM((_CLS_C, 16), jnp.float32),
        pltpu.VMEM((_CLS_C, 16), jnp.float32),
        pltpu.VMEM((_CLS_C,), jnp.float32),
        pltpu.VMEM((_CLS_C,), jnp.float32),
        pltpu.VMEM((16,), jnp.float32),
        pltpu.VMEM((16,), jnp.float32),
    ] + [pltpu.SemaphoreType.DMA] * 10,
)
def _cls_kernel(src_hbm, dst_hbm, p_hbm, q_hbm, w_hbm, b_hbm, out_hbm,
                si0, si1, di0, di1, pa0, pa1, qa0, qa1, o0, o1, w_v, b_v,
                isem0, isem1, dsem0, dsem1, psem0, psem1, qsem0, qsem1,
                osem0, osem1):
    c = lax.axis_index("c")
    s = lax.axis_index("s")

    pltpu.sync_copy(w_hbm, w_v)
    pltpu.sync_copy(b_hbm, b_v)
    wvec = w_v[...]
    ws = [wvec[f] for f in range(16)]
    bc2v = b_v[...]

    base = (c * NS + s) * _CLS_EPW
    NIT = _CLS_EPW // _CLS_C

    def src_sl(j):
        return src_hbm.at[pl.ds(base + j * _CLS_C, _CLS_C)]

    def dst_sl(j):
        return dst_hbm.at[pl.ds(base + j * _CLS_C, _CLS_C)]

    def out_sl(j):
        return out_hbm.at[pl.ds(base + j * _CLS_C, _CLS_C)]

    def compute(pa, qa, o):
        @pl.loop(0, _CLS_C // 16)
        def _(t):
            rowi = t * 16 + lax.iota(jnp.int32, 16)
            zero = jnp.zeros((16,), jnp.float32)
            accs = [zero, zero, zero, zero]
            for f in range(16):
                colf = jnp.full((16,), f, jnp.int32)
                av = plsc.load_gather(pa, [rowi, colf])
                bv = plsc.load_gather(qa, [rowi, colf])
                accs[f % 4] = accs[f % 4] \
                    + jnp.maximum(av + bv, 0.0) * ws[f]
            logit = (accs[0] + accs[1]) + (accs[2] + accs[3]) + bc2v
            o[pl.ds(t * 16, 16)] = 1.0 / (1.0 + jnp.exp(-logit))

    pltpu.sync_copy(src_sl(0), si0)
    pltpu.sync_copy(dst_sl(0), di0)
    pltpu.async_copy(p_hbm.at[si0], pa0, psem0)
    pltpu.async_copy(q_hbm.at[di0], qa0, qsem0)

    @pl.loop(0, NIT - 1, step=2)
    def _(i):
        pltpu.sync_copy(src_sl(i + 1), si1)
        pltpu.sync_copy(dst_sl(i + 1), di1)
        pltpu.make_async_copy(p_hbm.at[si0], pa0, psem0).wait()
        pltpu.make_async_copy(q_hbm.at[di0], qa0, qsem0).wait()
        pltpu.async_copy(p_hbm.at[si1], pa1, psem1)
        pltpu.async_copy(q_hbm.at[di1], qa1, qsem1)

        @pl.when(i > 0)
        def _():
            pltpu.make_async_copy(o0, out_sl(i - 2), osem0).wait()

        compute(pa0, qa0, o0)
        pltpu.async_copy(o0, out_sl(i), osem0)

        @pl.when(i + 2 < NIT - 1)
        def _():
            pltpu.sync_copy(src_sl(i + 2), si0)
            pltpu.sync_copy(dst_sl(i + 2), di0)

        pltpu.make_async_copy(p_hbm.at[si1], pa1, psem1).wait()
        pltpu.make_async_copy(q_hbm.at[di1], qa1, qsem1).wait()

        @pl.when(i + 2 < NIT - 1)
        def _():
            pltpu.async_copy(p_hbm.at[si0], pa0, psem0)
            pltpu.async_copy(q_hbm.at[di0], qa0, qsem0)

        @pl.when(i > 0)
        def _():
            pltpu.make_async_copy(o1, out_sl(i - 1), osem1).wait()

        compute(pa1, qa1, o1)
        pltpu.async_copy(o1, out_sl(i + 1), osem1)

    pltpu.make_async_copy(o0, out_sl(NIT - 3), osem0).wait()
    pltpu.make_async_copy(o1, out_sl(NIT - 2), osem1).wait()

    pltpu.sync_copy(src_sl(NIT - 1), si0)
    pltpu.sync_copy(dst_sl(NIT - 1), di0)
    pltpu.sync_copy(p_hbm.at[si0], pa0)
    pltpu.sync_copy(q_hbm.at[di0], qa0)
    compute(pa0, qa0, o0)
    pltpu.sync_copy(o0, out_sl(NIT - 1))


_R = 2000
_G = N // _R


def _tc_xw1(x, W1):
    def body(x_ref, w_ref, o_ref):
        o_ref[...] = jnp.dot(x_ref[...], w_ref[...],
                             preferred_element_type=jnp.float32)

    return pl.pallas_call(
        body,
        grid=(_G,),
        in_specs=[pl.BlockSpec((_R, 9), lambda i: (i, 0)),
                  pl.BlockSpec((9, 64), lambda i: (0, 0))],
        out_specs=pl.BlockSpec((_R, 64), lambda i: (i, 0)),
        out_shape=jax.ShapeDtypeStruct((N, 64), jnp.float32),
    )(x, W1)


def _tc_prep1(deg2, xw1):
    def body(d_ref, xw_ref, dinv_ref, y0_ref, y1_ref, y2_ref, y3_ref):
        deg = d_ref[0] + d_ref[1] + 1.0
        dv = lax.rsqrt(deg)
        y = dv * xw_ref[...]
        dinv_ref[...] = dv
        y0_ref[...] = y[:, 0:16]
        y1_ref[...] = y[:, 16:32]
        y2_ref[...] = y[:, 32:48]
        y3_ref[...] = y[:, 48:64]

    return pl.pallas_call(
        body,
        grid=(_G,),
        in_specs=[pl.BlockSpec((2, _R, 1), lambda i: (0, i, 0)),
                  pl.BlockSpec((_R, 64), lambda i: (i, 0))],
        out_specs=[pl.BlockSpec((_R, 1), lambda i: (i, 0))]
        + [pl.BlockSpec((_R, 16), lambda i: (i, 0))] * 4,
        out_shape=[jax.ShapeDtypeStruct((N, 1), jnp.float32)]
        + [jax.ShapeDtypeStruct((N, 16), jnp.float32)] * 4,
    )(deg2, xw1)


def _tc_mid(accs, ys, dinv, b1, W2):
    def body(a0, a1, a2, a3, y0, y1, y2r, y3, dv_ref, b_ref, w_ref,
             oa_ref, ob_ref):
        dv = dv_ref[...]
        b = b_ref[...]
        hs = [jnp.maximum(dv * (a[...] + y[...]) + b[:, 16 * q:16 * (q + 1)],
                          0.0)
              for q, (a, y) in enumerate(zip((a0, a1, a2, a3),
                                             (y0, y1, y2r, y3)))]
        h1 = jnp.concatenate(hs, axis=1)
        y2 = dv * jnp.dot(h1, w_ref[...], preferred_element_type=jnp.float32)
        oa_ref[...] = y2[:, :16]
        ob_ref[...] = y2[:, 16:]

    return pl.pallas_call(
        body,
        grid=(_G,),
        in_specs=[pl.BlockSpec((_R, 16), lambda i: (i, 0))] * 8
        + [pl.BlockSpec((_R, 1), lambda i: (i, 0)),
           pl.BlockSpec((1, 64), lambda i: (0, 0)),
           pl.BlockSpec((64, 32), lambda i: (0, 0))],
        out_specs=[pl.BlockSpec((_R, 16), lambda i: (i, 0)),
                   pl.BlockSpec((_R, 16), lambda i: (i, 0))],
        out_shape=[jax.ShapeDtypeStruct((N, 16), jnp.float32),
                   jax.ShapeDtypeStruct((N, 16), jnp.float32)],
    )(*accs, *ys, dinv, b1, W2)


def _tc_cls_prep(acc2a, acc2b, y2a, y2b, dinv, b2, Wc1, bc1):
    def body(aa_ref, ab_ref, ya_ref, yb_ref, dv_ref, b_ref, w_ref, bc_ref,
             p_ref, q_ref):
        dv = dv_ref[...]
        h2a = dv * (aa_ref[...] + ya_ref[...]) + b_ref[:, :16]
        h2b = dv * (ab_ref[...] + yb_ref[...]) + b_ref[:, 16:]
        h2 = jnp.concatenate([h2a, h2b], axis=1)
        w = w_ref[...]
        p_ref[...] = jnp.dot(h2, w[:32], preferred_element_type=jnp.float32) \
            + bc_ref[...]
        q_ref[...] = jnp.dot(h2, w[32:], preferred_element_type=jnp.float32)

    return pl.pallas_call(
        body,
        grid=(_G,),
        in_specs=[pl.BlockSpec((_R, 16), lambda i: (i, 0)),
                  pl.BlockSpec((_R, 16), lambda i: (i, 0)),
                  pl.BlockSpec((_R, 16), lambda i: (i, 0)),
                  pl.BlockSpec((_R, 16), lambda i: (i, 0)),
                  pl.BlockSpec((_R, 1), lambda i: (i, 0)),
                  pl.BlockSpec((1, 32), lambda i: (0, 0)),
                  pl.BlockSpec((64, 16), lambda i: (0, 0)),
                  pl.BlockSpec((1, 16), lambda i: (0, 0))],
        out_specs=[pl.BlockSpec((_R, 16), lambda i: (i, 0)),
                   pl.BlockSpec((_R, 16), lambda i: (i, 0))],
        out_shape=[jax.ShapeDtypeStruct((N, 16), jnp.float32),
                   jax.ShapeDtypeStruct((N, 16), jnp.float32)],
    )(acc2a, acc2b, y2a, y2b, dinv, b2, Wc1, bc1)


def kernel(x, edge_index, W1, b1, W2, b2, Wc1, bc1, Wc2, bc2):
    src_idx = edge_index[0]
    dst_idx = edge_index[1]
    deg2 = _deg_kernel(dst_idx)
    xw1 = _tc_xw1(x, W1)
    dinv, y10, y11, y12, y13 = _tc_prep1(deg2.reshape(2, N, 1), xw1)
    acc1s = _conv_scatter_q2(src_idx, dst_idx, y10, y11, y12, y13)
    y2a, y2b = _tc_mid(acc1s, (y10, y11, y12, y13), dinv,
                       b1.reshape(1, 64), W2)
    acc2a, acc2b = _conv_scatter_q1(src_idx, dst_idx, y2a, y2b)
    p, q = _tc_cls_prep(acc2a, acc2b, y2a, y2b, dinv,
                        b2.reshape(1, 32), Wc1, bc1.reshape(1, 16))
    scores = _cls_kernel(src_idx, dst_idx, p, q, Wc2.reshape(16),
                         jnp.broadcast_to(bc2, (16,)))
    return scores.reshape(E, 1)


# speedup vs baseline: 37.8244x; 1.1126x over previous
import functools

import jax
import jax.numpy as jnp
from jax import lax
from jax.experimental import pallas as pl
from jax.experimental.pallas import tpu as pltpu
from jax.experimental.pallas import tpu_sc as plsc

N = 50000
E = 1600000
NC = 2
NS = 16

_mesh = plsc.VectorSubcoreMesh(core_axis_name="c", subcore_axis_name="s")
_sc_params = pltpu.CompilerParams(use_tc_tiling_on_sc=False,
                                  needs_layout_passes=False)


_DEG_C = 2000
_DEG_EPW = E // (NC * NS)


@functools.partial(
    pl.kernel,
    out_type=jax.ShapeDtypeStruct((NC * N,), jnp.float32),
    mesh=_mesh,
    compiler_params=_sc_params,
    scratch_types=[
        pltpu.VMEM((_DEG_C,), jnp.int32),
        pltpu.VMEM((_DEG_C,), jnp.float32),
        pltpu.VMEM((5000,), jnp.float32),
        pltpu.VMEM_SHARED((N,), jnp.float32),
    ],
)
def _deg_kernel(dst_hbm, out_hbm, idx_v, ones_v, zero_v, acc_sh):
    c = lax.axis_index("c")
    s = lax.axis_index("s")

    @pl.loop(0, _DEG_C, step=16)
    def _(k):
        ones_v[pl.ds(k, 16)] = jnp.ones((16,), jnp.float32)

    @pl.loop(0, 5000, step=16)
    def _(k):
        zero_v[pl.ds(k, 16)] = jnp.zeros((16,), jnp.float32)

    @pl.when(s < 10)
    def _():
        pltpu.sync_copy(zero_v, acc_sh.at[pl.ds(s * 5000, 5000)])

    plsc.subcore_barrier()

    base = (c * NS + s) * _DEG_EPW

    @pl.loop(0, _DEG_EPW, step=_DEG_C)
    def _(i):
        pltpu.sync_copy(dst_hbm.at[pl.ds(base + i, _DEG_C)], idx_v)
        pltpu.sync_copy(ones_v, acc_sh.at[idx_v], add=True)

    plsc.subcore_barrier()

    @pl.when(s < 10)
    def _():
        pltpu.sync_copy(acc_sh.at[pl.ds(s * 5000, 5000)], zero_v)
        pltpu.sync_copy(zero_v, out_hbm.at[pl.ds(c * N + s * 5000, 5000)])


def _make_conv_scatter(NPC, C):
    H = 16
    EPW = E // NS
    ZR = 1000
    NZCH = N // ZR
    NQ = NC * NPC

    @functools.partial(
        pl.kernel,
        out_type=tuple(jax.ShapeDtypeStruct((N, H), jnp.float32)
                       for _ in range(NQ)),
        mesh=_mesh,
        compiler_params=_sc_params,
        scratch_types=[
            pltpu.VMEM((C,), jnp.int32),
            pltpu.VMEM((C,), jnp.int32),
            pltpu.VMEM((C,), jnp.int32),
            pltpu.VMEM((C,), jnp.int32),
            pltpu.VMEM((C, H), jnp.float32),
            pltpu.VMEM((C, H), jnp.float32),
            pltpu.VMEM((ZR, H), jnp.float32),
            pltpu.VMEM_SHARED((N, H), jnp.float32),
        ] + [pltpu.SemaphoreType.DMA] * 8,
    )
    def conv_kernel(src_hbm, dst_hbm, *refs):
        y_refs = refs[:NQ]
        out_refs = refs[NQ:2 * NQ]
        (si0, si1, di0, di1, rows0, rows1, zero_v, acc_sh,
         gsem0, gsem1, ssem0, ssem1, isem0, isem1, dsem0, dsem1) = refs[2 * NQ:]
        c = lax.axis_index("c")
        s = lax.axis_index("s")

        @pl.loop(0, ZR)
        def _(r):
            zero_v[r, pl.ds(0, 16)] = jnp.zeros((16,), jnp.float32)

        base = s * EPW

        def run_pass(y_hbm, out_hbm):
            for j in range((NZCH + NS - 1) // NS):
                k = s + j * NS

                @pl.when(k < NZCH)
                def _():
                    pltpu.sync_copy(zero_v, acc_sh.at[pl.ds(k * ZR, ZR)])

            plsc.subcore_barrier()

            NIT = EPW // C

            def src_sl(j):
                return src_hbm.at[pl.ds(base + j * C, C)]

            def dst_sl(j):
                return dst_hbm.at[pl.ds(base + j * C, C)]

            pltpu.async_copy(dst_sl(0), di0, dsem0)
            pltpu.sync_copy(src_sl(0), si0)
            pltpu.async_copy(y_hbm.at[si0], rows0, gsem0)
            pltpu.async_copy(src_sl(1), si1, isem1)
            pltpu.async_copy(dst_sl(1), di1, dsem1)

            @pl.loop(0, NIT, step=2)
            def _(i):
                pltpu.make_async_copy(y_hbm.at[si0], rows0, gsem0).wait()
                pltpu.make_async_copy(dst_sl(i), di0, dsem0).wait()
                pltpu.async_copy(rows0, acc_sh.at[di0], ssem0, add=True)

                @pl.when(i + 2 < NIT)
                def _():
                    pltpu.async_copy(src_sl(i + 2), si0, isem0)

                @pl.when(i > 0)
                def _():
                    pltpu.make_async_copy(rows1, acc_sh.at[di1], ssem1).wait()
                    pltpu.async_copy(dst_sl(i + 1), di1, dsem1)

                pltpu.make_async_copy(src_sl(i + 1), si1, isem1).wait()
                pltpu.async_copy(y_hbm.at[si1], rows1, gsem1)
                pltpu.make_async_copy(y_hbm.at[si1], rows1, gsem1).wait()
                pltpu.make_async_copy(dst_sl(i + 1), di1, dsem1).wait()
                pltpu.async_copy(rows1, acc_sh.at[di1], ssem1, add=True)

                @pl.when(i + 3 < NIT)
                def _():
                    pltpu.async_copy(src_sl(i + 3), si1, isem1)

                pltpu.make_async_copy(rows0, acc_sh.at[di0], ssem0).wait()

                @pl.when(i + 2 < NIT)
                def _():
                    pltpu.async_copy(dst_sl(i + 2), di0, dsem0)
                    pltpu.make_async_copy(src_sl(i + 2), si0, isem0).wait()
                    pltpu.async_copy(y_hbm.at[si0], rows0, gsem0)

            pltpu.make_async_copy(rows1, acc_sh.at[di1], ssem1).wait()

            plsc.subcore_barrier()

            for j in range((NZCH + NS - 1) // NS):
                k = s + j * NS

                @pl.when(k < NZCH)
                def _():
                    pltpu.sync_copy(acc_sh.at[pl.ds(k * ZR, ZR)], zero_v)
                    pltpu.sync_copy(zero_v, out_hbm.at[pl.ds(k * ZR, ZR)])

            plsc.subcore_barrier()

            @pl.loop(0, ZR)
            def _(r):
                zero_v[r, pl.ds(0, 16)] = jnp.zeros((16,), jnp.float32)

        for cv in range(NC):
            @pl.when(c == cv)
            def _(cv=cv):
                for p in range(NPC):
                    qi = cv * NPC + p
                    run_pass(y_refs[qi], out_refs[qi])

    return conv_kernel


_conv_scatter_q2 = _make_conv_scatter(2, 1000)
_conv_scatter_q1 = _make_conv_scatter(1, 1000)


_CLS_C = 400
_CLS_EPW = E // (NC * NS)


@functools.partial(
    pl.kernel,
    out_type=jax.ShapeDtypeStruct((E,), jnp.float32),
    mesh=_mesh,
    compiler_params=_sc_params,
    scratch_types=[
        pltpu.VMEM((_CLS_C,), jnp.int32),
        pltpu.VMEM((_CLS_C,), jnp.int32),
        pltpu.VMEM((_CLS_C,), jnp.int32),
        pltpu.VMEM((_CLS_C,), jnp.int32),
        pltpu.VMEM((_CLS_C, 16), jnp.float32),
        pltpu.VMEM((_CLS_C, 16), jnp.float32),
        pltpu.VMEM((_CLS_C, 16), jnp.float32),
        pltpu.VMEM((_CLS_C, 16), jnp.float32),
        pltpu.VMEM((_CLS_C,), jnp.float32),
        pltpu.VMEM((_CLS_C,), jnp.float32),
        pltpu.VMEM((16,), jnp.float32),
        pltpu.VMEM((16,), jnp.float32),
        pltpu.VMEM((32,), jnp.float32),
        pltpu.VMEM((32,), jnp.int32),
    ] + [pltpu.SemaphoreType.DMA] * 10,
)
def _cls_kernel(src_hbm, dst_hbm, p_hbm, q_hbm, w_hbm, b_hbm, out_hbm,
                si0, si1, di0, di1, pa0, pa1, qa0, qa1, o0, o1, w_v, b_v,
                wd_v, id2_v,
                isem0, isem1, dsem0, dsem1, psem0, psem1, qsem0, qsem1,
                osem0, osem1):
    c = lax.axis_index("c")
    s = lax.axis_index("s")

    pltpu.sync_copy(w_hbm, w_v)
    pltpu.sync_copy(b_hbm, b_v)
    wvec = w_v[...]
    bc2v = b_v[...]
    wd_v[pl.ds(0, 16)] = wvec
    wd_v[pl.ds(16, 16)] = wvec
    iot = lax.iota(jnp.int32, 16)
    id2_v[pl.ds(0, 16)] = iot
    id2_v[pl.ds(16, 16)] = iot
    rot_w = [wd_v[pl.ds(f, 16)] for f in range(16)]
    rot_i = [id2_v[pl.ds(f, 16)] for f in range(16)]

    base = (c * NS + s) * _CLS_EPW
    NIT = _CLS_EPW // _CLS_C

    def src_sl(j):
        return src_hbm.at[pl.ds(base + j * _CLS_C, _CLS_C)]

    def dst_sl(j):
        return dst_hbm.at[pl.ds(base + j * _CLS_C, _CLS_C)]

    def out_sl(j):
        return out_hbm.at[pl.ds(base + j * _CLS_C, _CLS_C)]

    def compute(pa, qa, o):
        @pl.loop(0, _CLS_C // 16)
        def _(t):
            rowi = t * 16 + lax.iota(jnp.int32, 16)
            zero = jnp.zeros((16,), jnp.float32)
            accs = [zero, zero, zero, zero]
            for f in range(16):
                colf = rot_i[f]
                av = plsc.load_gather(pa, [rowi, colf])
                bv = plsc.load_gather(qa, [rowi, colf])
                accs[f % 4] = accs[f % 4] \
                    + jnp.maximum(av + bv, 0.0) * rot_w[f]
            logit = (accs[0] + accs[1]) + (accs[2] + accs[3]) + bc2v
            o[pl.ds(t * 16, 16)] = 1.0 / (1.0 + jnp.exp(-logit))

    pltpu.sync_copy(src_sl(0), si0)
    pltpu.sync_copy(dst_sl(0), di0)
    pltpu.async_copy(p_hbm.at[si0], pa0, psem0)
    pltpu.async_copy(q_hbm.at[di0], qa0, qsem0)

    @pl.loop(0, NIT - 1, step=2)
    def _(i):
        pltpu.sync_copy(src_sl(i + 1), si1)
        pltpu.sync_copy(dst_sl(i + 1), di1)
        pltpu.make_async_copy(p_hbm.at[si0], pa0, psem0).wait()
        pltpu.make_async_copy(q_hbm.at[di0], qa0, qsem0).wait()
        pltpu.async_copy(p_hbm.at[si1], pa1, psem1)
        pltpu.async_copy(q_hbm.at[di1], qa1, qsem1)

        @pl.when(i > 0)
        def _():
            pltpu.make_async_copy(o0, out_sl(i - 2), osem0).wait()

        compute(pa0, qa0, o0)
        pltpu.async_copy(o0, out_sl(i), osem0)

        @pl.when(i + 2 < NIT - 1)
        def _():
            pltpu.sync_copy(src_sl(i + 2), si0)
            pltpu.sync_copy(dst_sl(i + 2), di0)

        pltpu.make_async_copy(p_hbm.at[si1], pa1, psem1).wait()
        pltpu.make_async_copy(q_hbm.at[di1], qa1, qsem1).wait()

        @pl.when(i + 2 < NIT - 1)
        def _():
            pltpu.async_copy(p_hbm.at[si0], pa0, psem0)
            pltpu.async_copy(q_hbm.at[di0], qa0, qsem0)

        @pl.when(i > 0)
        def _():
            pltpu.make_async_copy(o1, out_sl(i - 1), osem1).wait()

        compute(pa1, qa1, o1)
        pltpu.async_copy(o1, out_sl(i + 1), osem1)

    pltpu.make_async_copy(o0, out_sl(NIT - 3), osem0).wait()
    pltpu.make_async_copy(o1, out_sl(NIT - 2), osem1).wait()

    pltpu.sync_copy(src_sl(NIT - 1), si0)
    pltpu.sync_copy(dst_sl(NIT - 1), di0)
    pltpu.sync_copy(p_hbm.at[si0], pa0)
    pltpu.sync_copy(q_hbm.at[di0], qa0)
    compute(pa0, qa0, o0)
    pltpu.sync_copy(o0, out_sl(NIT - 1))


_R = 2000
_G = N // _R


def _tc_xw1(x, W1):
    def body(x_ref, w_ref, o_ref):
        o_ref[...] = jnp.dot(x_ref[...], w_ref[...],
                             preferred_element_type=jnp.float32)

    return pl.pallas_call(
        body,
        grid=(_G,),
        in_specs=[pl.BlockSpec((_R, 9), lambda i: (i, 0)),
                  pl.BlockSpec((9, 64), lambda i: (0, 0))],
        out_specs=pl.BlockSpec((_R, 64), lambda i: (i, 0)),
        out_shape=jax.ShapeDtypeStruct((N, 64), jnp.float32),
    )(x, W1)


def _tc_prep1(deg2, xw1):
    def body(d_ref, xw_ref, dinv_ref, y0_ref, y1_ref, y2_ref, y3_ref):
        deg = d_ref[0] + d_ref[1] + 1.0
        dv = lax.rsqrt(deg)
        y = dv * xw_ref[...]
        dinv_ref[...] = dv
        y0_ref[...] = y[:, 0:16]
        y1_ref[...] = y[:, 16:32]
        y2_ref[...] = y[:, 32:48]
        y3_ref[...] = y[:, 48:64]

    return pl.pallas_call(
        body,
        grid=(_G,),
        in_specs=[pl.BlockSpec((2, _R, 1), lambda i: (0, i, 0)),
                  pl.BlockSpec((_R, 64), lambda i: (i, 0))],
        out_specs=[pl.BlockSpec((_R, 1), lambda i: (i, 0))]
        + [pl.BlockSpec((_R, 16), lambda i: (i, 0))] * 4,
        out_shape=[jax.ShapeDtypeStruct((N, 1), jnp.float32)]
        + [jax.ShapeDtypeStruct((N, 16), jnp.float32)] * 4,
    )(deg2, xw1)


def _tc_mid(accs, ys, dinv, b1, W2):
    def body(a0, a1, a2, a3, y0, y1, y2r, y3, dv_ref, b_ref, w_ref,
             oa_ref, ob_ref):
        dv = dv_ref[...]
        b = b_ref[...]
        hs = [jnp.maximum(dv * (a[...] + y[...]) + b[:, 16 * q:16 * (q + 1)],
                          0.0)
              for q, (a, y) in enumerate(zip((a0, a1, a2, a3),
                                             (y0, y1, y2r, y3)))]
        h1 = jnp.concatenate(hs, axis=1)
        y2 = dv * jnp.dot(h1, w_ref[...], preferred_element_type=jnp.float32)
        oa_ref[...] = y2[:, :16]
        ob_ref[...] = y2[:, 16:]

    return pl.pallas_call(
        body,
        grid=(_G,),
        in_specs=[pl.BlockSpec((_R, 16), lambda i: (i, 0))] * 8
        + [pl.BlockSpec((_R, 1), lambda i: (i, 0)),
           pl.BlockSpec((1, 64), lambda i: (0, 0)),
           pl.BlockSpec((64, 32), lambda i: (0, 0))],
        out_specs=[pl.BlockSpec((_R, 16), lambda i: (i, 0)),
                   pl.BlockSpec((_R, 16), lambda i: (i, 0))],
        out_shape=[jax.ShapeDtypeStruct((N, 16), jnp.float32),
                   jax.ShapeDtypeStruct((N, 16), jnp.float32)],
    )(*accs, *ys, dinv, b1, W2)


def _tc_cls_prep(acc2a, acc2b, y2a, y2b, dinv, b2, Wc1, bc1):
    def body(aa_ref, ab_ref, ya_ref, yb_ref, dv_ref, b_ref, w_ref, bc_ref,
             p_ref, q_ref):
        dv = dv_ref[...]
        h2a = dv * (aa_ref[...] + ya_ref[...]) + b_ref[:, :16]
        h2b = dv * (ab_ref[...] + yb_ref[...]) + b_ref[:, 16:]
        h2 = jnp.concatenate([h2a, h2b], axis=1)
        w = w_ref[...]
        p_ref[...] = jnp.dot(h2, w[:32], preferred_element_type=jnp.float32) \
            + bc_ref[...]
        q_ref[...] = jnp.dot(h2, w[32:], preferred_element_type=jnp.float32)

    return pl.pallas_call(
        body,
        grid=(_G,),
        in_specs=[pl.BlockSpec((_R, 16), lambda i: (i, 0)),
                  pl.BlockSpec((_R, 16), lambda i: (i, 0)),
                  pl.BlockSpec((_R, 16), lambda i: (i, 0)),
                  pl.BlockSpec((_R, 16), lambda i: (i, 0)),
                  pl.BlockSpec((_R, 1), lambda i: (i, 0)),
                  pl.BlockSpec((1, 32), lambda i: (0, 0)),
                  pl.BlockSpec((64, 16), lambda i: (0, 0)),
                  pl.BlockSpec((1, 16), lambda i: (0, 0))],
        out_specs=[pl.BlockSpec((_R, 16), lambda i: (i, 0)),
                   pl.BlockSpec((_R, 16), lambda i: (i, 0))],
        out_shape=[jax.ShapeDtypeStruct((N, 16), jnp.float32),
                   jax.ShapeDtypeStruct((N, 16), jnp.float32)],
    )(acc2a, acc2b, y2a, y2b, dinv, b2, Wc1, bc1)


def kernel(x, edge_index, W1, b1, W2, b2, Wc1, bc1, Wc2, bc2):
    src_idx = edge_index[0]
    dst_idx = edge_index[1]
    deg2 = _deg_kernel(dst_idx)
    xw1 = _tc_xw1(x, W1)
    dinv, y10, y11, y12, y13 = _tc_prep1(deg2.reshape(2, N, 1), xw1)
    acc1s = _conv_scatter_q2(src_idx, dst_idx, y10, y11, y12, y13)
    y2a, y2b = _tc_mid(acc1s, (y10, y11, y12, y13), dinv,
                       b1.reshape(1, 64), W2)
    acc2a, acc2b = _conv_scatter_q1(src_idx, dst_idx, y2a, y2b)
    p, q = _tc_cls_prep(acc2a, acc2b, y2a, y2b, dinv,
                        b2.reshape(1, 32), Wc1, bc1.reshape(1, 16))
    scores = _cls_kernel(src_idx, dst_idx, p, q, Wc2.reshape(16),
                         jnp.broadcast_to(bc2, (16,)))
    return scores.reshape(E, 1)
